# baseline (speedup 1.0000x reference)
import functools

import jax
import jax.numpy as jnp
from jax import lax
from jax.experimental import pallas as pl
from jax.experimental.pallas import tpu as pltpu
from jax.experimental.pallas import tpu_sc as plsc

NUM_CLASSES = 26
BATCH = 16384
NUM_CORES = 2
NUM_SUBCORES = 16
LANES = 16
NUM_WORKERS = NUM_CORES * NUM_SUBCORES
ROWS_PER_W = BATCH // NUM_WORKERS
GROUPS = ROWS_PER_W // LANES
FLAT_PER_W = ROWS_PER_W * NUM_CLASSES
NUM_CHUNKS = 4
GROUPS_PER_CHUNK = GROUPS // NUM_CHUNKS
WORDS_PER_CHUNK = FLAT_PER_W // NUM_CHUNKS


@functools.partial(
    pl.kernel,
    out_type=jax.ShapeDtypeStruct((BATCH,), jnp.int32),
    mesh=plsc.VectorSubcoreMesh(core_axis_name="c", subcore_axis_name="s"),
    compiler_params=pltpu.CompilerParams(needs_layout_passes=False),
    scratch_types=[
        pltpu.VMEM((FLAT_PER_W,), jnp.float32),
        pltpu.VMEM((NUM_CLASSES,), jnp.int32),
        pltpu.VMEM((ROWS_PER_W,), jnp.int32),
        [pltpu.SemaphoreType.DMA] * NUM_CHUNKS,
    ],
)
def _prob_to_label_sc(in_hbm, tab_hbm, out_hbm, vals_v, tab_v, out_v, sems):
    wid = lax.axis_index("s") * NUM_CORES + lax.axis_index("c")
    base_word = wid * FLAT_PER_W
    base_row = wid * ROWS_PER_W

    copies = [
        pltpu.async_copy(
            in_hbm.at[pl.ds(base_word + k * WORDS_PER_CHUNK, WORDS_PER_CHUNK)],
            vals_v.at[pl.ds(k * WORDS_PER_CHUNK, WORDS_PER_CHUNK)],
            sems[k],
        )
        for k in range(NUM_CHUNKS)
    ]
    pltpu.sync_copy(tab_hbm, tab_v)

    lane = lax.iota(jnp.int32, LANES)
    lane_off = lane * NUM_CLASSES
    consts = [jnp.full((LANES,), c, jnp.int32) for c in range(NUM_CLASSES)]
    offs = [lane_off + c for c in range(NUM_CLASSES)]

    def body(g):
        base = jnp.full((LANES,), g * (LANES * NUM_CLASSES), jnp.int32)
        items = [
            (plsc.load_gather(vals_v, [base + offs[c]]), consts[c])
            for c in range(NUM_CLASSES)
        ]
        while len(items) > 1:
            nxt = []
            for i in range(0, len(items) - 1, 2):
                (va, ia), (vb, ib) = items[i], items[i + 1]
                upd = vb > va
                nxt.append((jnp.where(upd, vb, va), jnp.where(upd, ib, ia)))
            if len(items) % 2:
                nxt.append(items[-1])
            items = nxt
        best_v, best_i = items[0]
        labels = plsc.load_gather(tab_v, [best_i])
        out_v[pl.ds(g * LANES, LANES)] = labels

    for k in range(NUM_CHUNKS):
        copies[k].wait()
        plsc.parallel_loop(
            k * GROUPS_PER_CHUNK, (k + 1) * GROUPS_PER_CHUNK, unroll=2
        )(body)

    pltpu.sync_copy(out_v, out_hbm.at[pl.ds(base_row, ROWS_PER_W)])


def kernel(inputs, label_table):
    return _prob_to_label_sc(inputs.reshape(-1), label_table)


# speedup vs baseline: 1.0636x; 1.0636x over previous
import functools

import jax
import jax.numpy as jnp
from jax import lax
from jax.experimental import pallas as pl
from jax.experimental.pallas import tpu as pltpu
from jax.experimental.pallas import tpu_sc as plsc

NUM_CLASSES = 26
BATCH = 16384
NUM_CORES = 2
NUM_SUBCORES = 16
LANES = 16
NUM_WORKERS = NUM_CORES * NUM_SUBCORES
ROWS_PER_W = BATCH // NUM_WORKERS
GROUPS = ROWS_PER_W // LANES
NUM_CHUNKS = 4
ROWS_PER_CHUNK = ROWS_PER_W // NUM_CHUNKS
GROUPS_PER_CHUNK = GROUPS // NUM_CHUNKS


@functools.partial(
    pl.kernel,
    out_type=jax.ShapeDtypeStruct((BATCH,), jnp.int32),
    mesh=plsc.VectorSubcoreMesh(core_axis_name="c", subcore_axis_name="s"),
    compiler_params=pltpu.CompilerParams(needs_layout_passes=False),
    scratch_types=[
        pltpu.VMEM((ROWS_PER_W, NUM_CLASSES), jnp.float32),
        pltpu.VMEM((NUM_CLASSES,), jnp.int32),
        pltpu.VMEM((ROWS_PER_W,), jnp.int32),
        [pltpu.SemaphoreType.DMA] * NUM_CHUNKS,
    ],
)
def _prob_to_label_sc(in_hbm, tab_hbm, out_hbm, vals_v, tab_v, out_v, sems):
    wid = lax.axis_index("s") * NUM_CORES + lax.axis_index("c")
    base_row = wid * ROWS_PER_W

    copies = [
        pltpu.async_copy(
            in_hbm.at[pl.ds(base_row + k * ROWS_PER_CHUNK, ROWS_PER_CHUNK), :],
            vals_v.at[pl.ds(k * ROWS_PER_CHUNK, ROWS_PER_CHUNK), :],
            sems[k],
        )
        for k in range(NUM_CHUNKS)
    ]
    pltpu.sync_copy(tab_hbm, tab_v)

    lane = lax.iota(jnp.int32, LANES)
    consts = [jnp.full((LANES,), c, jnp.int32) for c in range(NUM_CLASSES)]

    def do_group(g):
        sub = vals_v.at[pl.ds(g * LANES, LANES), :]
        items = [
            (plsc.load_gather(sub, [lane, consts[c]]), consts[c])
            for c in range(NUM_CLASSES)
        ]
        while len(items) > 1:
            nxt = []
            for i in range(0, len(items) - 1, 2):
                (va, ia), (vb, ib) = items[i], items[i + 1]
                upd = vb > va
                nxt.append((jnp.maximum(va, vb), jnp.where(upd, ib, ia)))
            if len(items) % 2:
                nxt.append(items[-1])
            items = nxt
        _, best_i = items[0]
        labels = plsc.load_gather(tab_v, [best_i])
        out_v[pl.ds(g * LANES, LANES)] = labels

    for k in range(NUM_CHUNKS):
        copies[k].wait()
        for g in range(k * GROUPS_PER_CHUNK, (k + 1) * GROUPS_PER_CHUNK):
            do_group(g)

    pltpu.sync_copy(out_v, out_hbm.at[pl.ds(base_row, ROWS_PER_W)])


def kernel(inputs, label_table):
    return _prob_to_label_sc(inputs, label_table)


# speedup vs baseline: 1.0785x; 1.0140x over previous
import functools

import jax
import jax.numpy as jnp
from jax import lax
from jax.experimental import pallas as pl
from jax.experimental.pallas import tpu as pltpu
from jax.experimental.pallas import tpu_sc as plsc

NUM_CLASSES = 26
ROW_PITCH = 26
BATCH = 16384
NUM_CORES = 2
NUM_SUBCORES = 16
LANES = 16
NUM_WORKERS = NUM_CORES * NUM_SUBCORES
ROWS_PER_W = BATCH // NUM_WORKERS
GROUPS = ROWS_PER_W // LANES
NUM_CHUNKS = 4
ROWS_PER_CHUNK = ROWS_PER_W // NUM_CHUNKS
GROUPS_PER_CHUNK = GROUPS // NUM_CHUNKS


@functools.partial(
    pl.kernel,
    out_type=jax.ShapeDtypeStruct((BATCH,), jnp.int32),
    mesh=plsc.VectorSubcoreMesh(core_axis_name="c", subcore_axis_name="s"),
    compiler_params=pltpu.CompilerParams(needs_layout_passes=False),
    scratch_types=[
        pltpu.VMEM((ROWS_PER_W, ROW_PITCH), jnp.float32),
        pltpu.VMEM((NUM_CLASSES,), jnp.int32),
        pltpu.VMEM((ROWS_PER_W,), jnp.int32),
        [pltpu.SemaphoreType.DMA] * NUM_CHUNKS,
    ],
)
def _prob_to_label_sc(in_hbm, tab_hbm, out_hbm, vals_v, tab_v, out_v, sems):
    wid = lax.axis_index("s") * NUM_CORES + lax.axis_index("c")
    base_row = wid * ROWS_PER_W

    copies = [
        pltpu.async_copy(
            in_hbm.at[pl.ds(base_row + k * ROWS_PER_CHUNK, ROWS_PER_CHUNK), :],
            vals_v.at[pl.ds(k * ROWS_PER_CHUNK, ROWS_PER_CHUNK), 0:NUM_CLASSES],
            sems[k],
        )
        for k in range(NUM_CHUNKS)
    ]
    pltpu.sync_copy(tab_hbm, tab_v)

    lane = lax.iota(jnp.int32, LANES)
    consts = [jnp.full((LANES,), c, jnp.int32) for c in range(NUM_CLASSES)]

    def body(g):
        rows = g * LANES + lane
        items = [
            (plsc.load_gather(vals_v, [rows, consts[c]]), consts[c])
            for c in range(NUM_CLASSES)
        ]
        while len(items) > 1:
            nxt = []
            for i in range(0, len(items) - 1, 2):
                (va, ia), (vb, ib) = items[i], items[i + 1]
                upd = vb > va
                nxt.append((jnp.where(upd, vb, va), jnp.where(upd, ib, ia)))
            if len(items) % 2:
                nxt.append(items[-1])
            items = nxt
        best_v, best_i = items[0]
        labels = plsc.load_gather(tab_v, [best_i])
        out_v[pl.ds(g * LANES, LANES)] = labels

    for k in range(NUM_CHUNKS):
        with jax.named_scope(f"wait{k}"):
            copies[k].wait()
        with jax.named_scope(f"comp{k}"):
            plsc.parallel_loop(
                k * GROUPS_PER_CHUNK, (k + 1) * GROUPS_PER_CHUNK, unroll=2
            )(body)

    with jax.named_scope("outcopy"):
        pltpu.sync_copy(out_v, out_hbm.at[pl.ds(base_row, ROWS_PER_W)])


def kernel(inputs, label_table):
    return _prob_to_label_sc(inputs, label_table)


# speedup vs baseline: 1.0857x; 1.0066x over previous
import functools

import jax
import jax.numpy as jnp
from jax import lax
from jax.experimental import pallas as pl
from jax.experimental.pallas import tpu as pltpu
from jax.experimental.pallas import tpu_sc as plsc

NUM_CLASSES = 26
ROW_PITCH = 26
BATCH = 16384
NUM_CORES = 2
NUM_SUBCORES = 16
LANES = 16
NUM_WORKERS = NUM_CORES * NUM_SUBCORES
ROWS_PER_W = BATCH // NUM_WORKERS
GROUPS = ROWS_PER_W // LANES
NUM_CHUNKS = 4
ROWS_PER_CHUNK = ROWS_PER_W // NUM_CHUNKS
GROUPS_PER_CHUNK = GROUPS // NUM_CHUNKS


@functools.partial(
    pl.kernel,
    out_type=jax.ShapeDtypeStruct((BATCH,), jnp.int32),
    mesh=plsc.VectorSubcoreMesh(core_axis_name="c", subcore_axis_name="s"),
    compiler_params=pltpu.CompilerParams(needs_layout_passes=False, use_tc_tiling_on_sc=True),
    scratch_types=[
        pltpu.VMEM((ROWS_PER_W, ROW_PITCH), jnp.float32),
        pltpu.VMEM((NUM_CLASSES,), jnp.int32),
        pltpu.VMEM((ROWS_PER_W,), jnp.int32),
        [pltpu.SemaphoreType.DMA] * NUM_CHUNKS,
    ],
)
def _prob_to_label_sc(in_hbm, tab_hbm, out_hbm, vals_v, tab_v, out_v, sems):
    wid = lax.axis_index("s") * NUM_CORES + lax.axis_index("c")
    base_row = wid * ROWS_PER_W

    copies = [
        pltpu.async_copy(
            in_hbm.at[pl.ds(base_row + k * ROWS_PER_CHUNK, ROWS_PER_CHUNK), :],
            vals_v.at[pl.ds(k * ROWS_PER_CHUNK, ROWS_PER_CHUNK), 0:NUM_CLASSES],
            sems[k],
        )
        for k in range(NUM_CHUNKS)
    ]
    pltpu.sync_copy(tab_hbm, tab_v)

    lane = lax.iota(jnp.int32, LANES)
    consts = [jnp.full((LANES,), c, jnp.int32) for c in range(NUM_CLASSES)]

    def body(g):
        rows = g * LANES + lane
        items = [
            (plsc.load_gather(vals_v, [rows, consts[c]]), consts[c])
            for c in range(NUM_CLASSES)
        ]
        while len(items) > 1:
            nxt = []
            for i in range(0, len(items) - 1, 2):
                (va, ia), (vb, ib) = items[i], items[i + 1]
                upd = vb > va
                nxt.append((jnp.where(upd, vb, va), jnp.where(upd, ib, ia)))
            if len(items) % 2:
                nxt.append(items[-1])
            items = nxt
        best_v, best_i = items[0]
        labels = plsc.load_gather(tab_v, [best_i])
        out_v[pl.ds(g * LANES, LANES)] = labels

    for k in range(NUM_CHUNKS):
        copies[k].wait()
        plsc.parallel_loop(
            k * GROUPS_PER_CHUNK, (k + 1) * GROUPS_PER_CHUNK, unroll=2
        )(body)

    pltpu.sync_copy(out_v, out_hbm.at[pl.ds(base_row, ROWS_PER_W)])


def kernel(inputs, label_table):
    return _prob_to_label_sc(inputs, label_table)


# speedup vs baseline: 1.1113x; 1.0236x over previous
import functools

import jax
import jax.numpy as jnp
from jax import lax
from jax.experimental import pallas as pl
from jax.experimental.pallas import tpu as pltpu
from jax.experimental.pallas import tpu_sc as plsc

NUM_CLASSES = 26
BATCH = 16384
NUM_CORES = 2
NUM_SUBCORES = 16
LANES = 16
NUM_WORKERS = NUM_CORES * NUM_SUBCORES
ROWS_PER_W = BATCH // NUM_WORKERS
GROUPS = ROWS_PER_W // LANES
CHUNK_ROWS = ((0, 64), (64, 64), (128, 128), (256, 256))
NUM_CHUNKS = len(CHUNK_ROWS)


@functools.partial(
    pl.kernel,
    out_type=jax.ShapeDtypeStruct((BATCH,), jnp.int32),
    mesh=plsc.VectorSubcoreMesh(core_axis_name="c", subcore_axis_name="s"),
    compiler_params=pltpu.CompilerParams(needs_layout_passes=False),
    scratch_types=[
        pltpu.VMEM((ROWS_PER_W, NUM_CLASSES), jnp.float32),
        pltpu.VMEM((NUM_CLASSES,), jnp.int32),
        pltpu.VMEM((ROWS_PER_W,), jnp.int32),
        [pltpu.SemaphoreType.DMA] * NUM_CHUNKS,
    ],
)
def _prob_to_label_sc(in_hbm, tab_hbm, out_hbm, vals_v, tab_v, out_v, sems):
    wid = lax.axis_index("s") * NUM_CORES + lax.axis_index("c")
    base_row = wid * ROWS_PER_W

    copies = []
    for k, (r0, nr) in enumerate(CHUNK_ROWS):
        copies.append(pltpu.async_copy(
            in_hbm.at[pl.ds(base_row + r0, nr), :],
            vals_v.at[pl.ds(r0, nr), :],
            sems[k],
        ))
    pltpu.sync_copy(tab_hbm, tab_v)

    lane = lax.iota(jnp.int32, LANES)
    consts = [jnp.full((LANES,), c, jnp.int32) for c in range(NUM_CLASSES)]

    def body(g):
        rows = g * LANES + lane
        items = [
            (plsc.load_gather(vals_v, [rows, consts[c]]), consts[c])
            for c in range(NUM_CLASSES)
        ]
        while len(items) > 1:
            nxt = []
            for i in range(0, len(items) - 1, 2):
                (va, ia), (vb, ib) = items[i], items[i + 1]
                upd = vb > va
                nxt.append((jnp.maximum(va, vb), jnp.where(upd, ib, ia)))
            if len(items) % 2:
                nxt.append(items[-1])
            items = nxt
        _, best_i = items[0]
        labels = plsc.load_gather(tab_v, [best_i])
        out_v[pl.ds(g * LANES, LANES)] = labels

    for k, (r0, nr) in enumerate(CHUNK_ROWS):
        copies[k].wait()
        plsc.parallel_loop(r0 // LANES, (r0 + nr) // LANES, unroll=2)(body)

    pltpu.sync_copy(out_v, out_hbm.at[pl.ds(base_row, ROWS_PER_W)])


def kernel(inputs, label_table):
    return _prob_to_label_sc(inputs, label_table)
